# baseline (speedup 1.0000x reference)
import functools

import jax
import jax.numpy as jnp
from jax import lax
from jax.experimental import pallas as pl
from jax.experimental.pallas import tpu as pltpu
from jax.experimental.pallas import tpu_sc as plsc

B = 4096
T = 200
L = 50
D = 64
N = B * L
V = B * T

NW = 32
PER_W = N // NW
CHUNK = 128
NCHUNK = PER_W // CHUNK
LANES = 16


def _body(idx_hbm, params_hbm, out_hbm, idx_v, gidx_v, rows_v, sem):
    wid = lax.axis_index("s") * 2 + lax.axis_index("c")
    wbase = wid * PER_W

    iota = lax.iota(jnp.int32, LANES)
    vl = jnp.full((LANES,), L, jnp.int32)
    vt = jnp.full((LANES,), T, jnp.int32)

    def step(j, _):
        base = wbase + j * CHUNK
        pltpu.sync_copy(idx_hbm.at[pl.ds(base, CHUNK)], idx_v)
        for i in range(CHUNK // LANES):
            p = jnp.full((LANES,), base + i * LANES, jnp.int32) + iota
            row = lax.div(p, vl)
            gidx_v[pl.ds(i * LANES, LANES)] = (
                idx_v[pl.ds(i * LANES, LANES)] + row * vt
            )
        pltpu.async_copy(params_hbm.at[gidx_v], rows_v, sem).wait()
        pltpu.sync_copy(rows_v, out_hbm.at[pl.ds(base, CHUNK)])
        return _

    lax.fori_loop(0, NCHUNK, step, None)


def kernel(params, indices):
    params_flat = params.reshape(V, D)
    idx_flat = indices.reshape(N).astype(jnp.int32)

    mesh = plsc.VectorSubcoreMesh(core_axis_name="c", subcore_axis_name="s")
    k = pl.kernel(
        _body,
        mesh=mesh,
        out_type=jax.ShapeDtypeStruct((N, D), jnp.float32),
        scratch_types=[
            pltpu.VMEM((CHUNK,), jnp.int32),
            pltpu.VMEM((CHUNK,), jnp.int32),
            pltpu.VMEM((CHUNK, D), jnp.float32),
            pltpu.SemaphoreType.DMA,
        ],
        compiler_params=pltpu.CompilerParams(use_tc_tiling_on_sc=False),
    )
    out = k(idx_flat, params_flat)
    return out.reshape(B, L, D)


# speedup vs baseline: 1.0918x; 1.0918x over previous
import jax
import jax.numpy as jnp
from jax import lax
from jax.experimental import pallas as pl
from jax.experimental.pallas import tpu as pltpu
from jax.experimental.pallas import tpu_sc as plsc

B = 4096
T = 200
L = 50
D = 64
N = B * L
V = B * T

NW = 32
PER_W = N // NW
CHUNK = 128
NCHUNK = PER_W // CHUNK
NBUF = 5
NGROUP = NCHUNK // NBUF
LANES = 16


def _body(idx_hbm, params_hbm, out_hbm,
          idx_all, gidx, rows, gsem, ssem):
    wid = lax.axis_index("s") * 2 + lax.axis_index("c")
    wbase = wid * PER_W

    pltpu.sync_copy(idx_hbm.at[pl.ds(wbase, PER_W)], idx_all)

    iota = lax.iota(jnp.int32, LANES)
    vl = jnp.full((LANES,), L, jnp.int32)
    vt = jnp.full((LANES,), T, jnp.int32)

    def fire(g, s):
        local = (g * NBUF + s) * CHUNK
        for i in range(CHUNK // LANES):
            p = jnp.full((LANES,), wbase + local + i * LANES, jnp.int32) + iota
            row = lax.div(p, vl)
            gidx[s, pl.ds(i * LANES, LANES)] = (
                idx_all[pl.ds(local + i * LANES, LANES)] + row * vt
            )
        pltpu.async_copy(params_hbm.at[gidx.at[s]], rows.at[s], gsem.at[s])

    def group(g, _):
        for s in range(NBUF):
            @pl.when(g > 0)
            def _wait_free():
                pltpu.make_async_copy(rows.at[s],
                                      out_hbm.at[pl.ds(0, CHUNK)],
                                      ssem.at[s]).wait()
            fire(g, s)
        for s in range(NBUF):
            pltpu.make_async_copy(params_hbm.at[gidx.at[s]], rows.at[s],
                                  gsem.at[s]).wait()
            base = wbase + (g * NBUF + s) * CHUNK
            pltpu.async_copy(rows.at[s], out_hbm.at[pl.ds(base, CHUNK)],
                             ssem.at[s])
        return _

    lax.fori_loop(0, NGROUP, group, None)
    for s in range(NBUF):
        pltpu.make_async_copy(rows.at[s], out_hbm.at[pl.ds(0, CHUNK)],
                              ssem.at[s]).wait()


def kernel(params, indices):
    params_flat = params.reshape(V, D)
    idx_flat = indices.reshape(N).astype(jnp.int32)

    mesh = plsc.VectorSubcoreMesh(core_axis_name="c", subcore_axis_name="s")
    k = pl.kernel(
        _body,
        mesh=mesh,
        out_type=jax.ShapeDtypeStruct((N, D), jnp.float32),
        scratch_types=[
            pltpu.VMEM((PER_W,), jnp.int32),
            pltpu.VMEM((NBUF, CHUNK), jnp.int32),
            pltpu.VMEM((NBUF, CHUNK, D), jnp.float32),
            pltpu.SemaphoreType.DMA((NBUF,)),
            pltpu.SemaphoreType.DMA((NBUF,)),
        ],
        compiler_params=pltpu.CompilerParams(use_tc_tiling_on_sc=False),
    )
    out = k(idx_flat, params_flat)
    return out.reshape(B, L, D)


# speedup vs baseline: 3.3566x; 3.0742x over previous
import jax
import jax.numpy as jnp
from jax import lax
from jax.experimental import pallas as pl
from jax.experimental.pallas import tpu as pltpu
from jax.experimental.pallas import tpu_sc as plsc

B = 4096
T = 200
L = 50
D = 64

NW = 32
NL = B // NW
NG = NL // 16
DC = 2
NDCH = D // DC
LANES = 16


def _body(pt_hbm, it_hbm, ot_hbm, idx_v, slab_v, out_v, gsem, ssem):
    wid = lax.axis_index("s") * 2 + lax.axis_index("c")
    b0 = wid * NL

    pltpu.sync_copy(it_hbm.at[:, pl.ds(b0, NL)], idx_v)

    iota = lax.iota(jnp.int32, LANES)
    lanes = [jnp.full((LANES,), g * LANES, jnp.int32) + iota for g in range(NG)]
    dsplat = [jnp.full((LANES,), d, jnp.int32) for d in range(DC)]

    def fire(c, s):
        pltpu.async_copy(
            pt_hbm.at[:, pl.ds(c * DC, DC), pl.ds(b0, NL)],
            slab_v.at[s], gsem.at[s])

    fire(0, 0)
    fire(1, 1)

    def chunk(c, s):
        pltpu.make_async_copy(
            pt_hbm.at[:, pl.ds(0, DC), pl.ds(b0, NL)],
            slab_v.at[s], gsem.at[s]).wait()

        @pl.when(c > 0)
        def _wait_out_free():
            pltpu.make_async_copy(
                out_v, ot_hbm.at[:, pl.ds(0, DC), pl.ds(b0, NL)],
                ssem).wait()

        def per_l(l, _):
            for g in range(NG):
                t = idx_v[l, pl.ds(g * LANES, LANES)]
                for d in range(DC):
                    v = plsc.load_gather(slab_v.at[s], [t, dsplat[d], lanes[g]])
                    out_v[l, d, pl.ds(g * LANES, LANES)] = v
            return _

        lax.fori_loop(0, L, per_l, None)
        pltpu.async_copy(out_v, ot_hbm.at[:, pl.ds(c * DC, DC), pl.ds(b0, NL)],
                         ssem)

        @pl.when(c + 2 < NDCH)
        def _refill():
            fire(c + 2, s)

    def group(g2, _):
        chunk(g2 * 2, 0)
        chunk(g2 * 2 + 1, 1)
        return _

    lax.fori_loop(0, NDCH // 2, group, None)
    pltpu.make_async_copy(
        out_v, ot_hbm.at[:, pl.ds(0, DC), pl.ds(b0, NL)], ssem).wait()


def kernel(params, indices):
    pt = params.transpose(1, 2, 0)
    it = indices.astype(jnp.int32).T

    mesh = plsc.VectorSubcoreMesh(core_axis_name="c", subcore_axis_name="s")
    k = pl.kernel(
        _body,
        mesh=mesh,
        out_type=jax.ShapeDtypeStruct((L, D, B), jnp.float32),
        scratch_types=[
            pltpu.VMEM((L, NL), jnp.int32),
            pltpu.VMEM((2, T, DC, NL), jnp.float32),
            pltpu.VMEM((L, DC, NL), jnp.float32),
            pltpu.SemaphoreType.DMA((2,)),
            pltpu.SemaphoreType.DMA,
        ],
        compiler_params=pltpu.CompilerParams(use_tc_tiling_on_sc=True,
                                             needs_layout_passes=False),
    )
    ot = k(pt, it)
    return ot.transpose(2, 0, 1)


# speedup vs baseline: 6.2754x; 1.8696x over previous
import jax
import jax.numpy as jnp
from jax import lax
from jax.experimental import pallas as pl
from jax.experimental.pallas import tpu as pltpu
from jax.experimental.pallas import tpu_sc as plsc

B = 4096
T = 200
L = 50
D = 64

NW = 32
NL = B // NW
NG = NL // 16
DC = 2
NDCH = D // DC
LANES = 16


def _body(pt_hbm, it_hbm, ot_hbm, idx_v, slab_v, out_v, gsem, ssem):
    wid = lax.axis_index("s") * 2 + lax.axis_index("c")
    b0 = wid * NL

    pltpu.sync_copy(it_hbm.at[:, pl.ds(b0, NL)], idx_v)

    iota = lax.iota(jnp.int32, LANES)
    lanes = [jnp.full((LANES,), g * LANES, jnp.int32) + iota for g in range(NG)]
    zero = jnp.zeros((LANES,), jnp.int32)
    tstride = jnp.full((LANES,), DC * NL, jnp.int32)
    dstride = jnp.full((LANES,), NL, jnp.int32)

    @plsc.parallel_loop(0, L, unroll=2)
    def _pre(l):
        for g in range(NG):
            sl = pl.ds(g * LANES, LANES)
            idx_v[l, sl] = idx_v[l, sl] * tstride + lanes[g]

    def fire(c, s):
        pltpu.async_copy(
            pt_hbm.at[:, pl.ds(c * DC, DC), pl.ds(b0, NL)],
            slab_v.at[s], gsem.at[s])

    fire(0, 0)
    fire(1, 1)

    def chunk(c, s):
        pltpu.make_async_copy(
            pt_hbm.at[:, pl.ds(0, DC), pl.ds(b0, NL)],
            slab_v.at[s], gsem.at[s]).wait()

        @pl.when(c > 0)
        def _wait_out_free():
            pltpu.make_async_copy(
                out_v, ot_hbm.at[:, pl.ds(0, DC), pl.ds(b0, NL)],
                ssem).wait()

        @plsc.parallel_loop(0, L, unroll=2)
        def per_l(l):
            for g in range(NG):
                sl = pl.ds(g * LANES, LANES)
                sidx = idx_v[l, sl]
                for d in range(DC):
                    v = plsc.load_gather(
                        slab_v.at[s],
                        [zero, zero, sidx if d == 0 else sidx + d * dstride])
                    out_v[l, d, sl] = v
        pltpu.async_copy(out_v, ot_hbm.at[:, pl.ds(c * DC, DC), pl.ds(b0, NL)],
                         ssem)

        @pl.when(c + 2 < NDCH)
        def _refill():
            fire(c + 2, s)

    def group(g2, _):
        chunk(g2 * 2, 0)
        chunk(g2 * 2 + 1, 1)
        return _

    lax.fori_loop(0, NDCH // 2, group, None)
    pltpu.make_async_copy(
        out_v, ot_hbm.at[:, pl.ds(0, DC), pl.ds(b0, NL)], ssem).wait()


def kernel(params, indices):
    pt = params.transpose(1, 2, 0)
    it = indices.astype(jnp.int32).T

    mesh = plsc.VectorSubcoreMesh(core_axis_name="c", subcore_axis_name="s")
    k = pl.kernel(
        _body,
        mesh=mesh,
        out_type=jax.ShapeDtypeStruct((L, D, B), jnp.float32),
        scratch_types=[
            pltpu.VMEM((L, NL), jnp.int32),
            pltpu.VMEM((2, T, DC, NL), jnp.float32),
            pltpu.VMEM((L, DC, NL), jnp.float32),
            pltpu.SemaphoreType.DMA((2,)),
            pltpu.SemaphoreType.DMA,
        ],
        compiler_params=pltpu.CompilerParams(use_tc_tiling_on_sc=True,
                                             needs_layout_passes=False),
    )
    ot = k(pt, it)
    return ot.transpose(2, 0, 1)


# speedup vs baseline: 6.2827x; 1.0012x over previous
import jax
import jax.numpy as jnp
from jax import lax
from jax.experimental import pallas as pl
from jax.experimental.pallas import tpu as pltpu
from jax.experimental.pallas import tpu_sc as plsc

B = 4096
T = 200
L = 50
D = 64

NW = 32
NL = B // NW
NG = NL // 16
DC = 2
NDCH = D // DC
LANES = 16


def _body(pt_hbm, it_hbm, ot_hbm, idx_v, slab_v, out_v, gsem, ssem):
    wid = lax.axis_index("s") * 2 + lax.axis_index("c")
    b0 = wid * NL

    pltpu.sync_copy(it_hbm.at[:, pl.ds(b0, NL)], idx_v)

    iota = lax.iota(jnp.int32, LANES)
    lanes = [jnp.full((LANES,), g * LANES, jnp.int32) + iota for g in range(NG)]
    zero = jnp.zeros((LANES,), jnp.int32)
    tstride = jnp.full((LANES,), DC * NL, jnp.int32)
    dstride = jnp.full((LANES,), NL, jnp.int32)

    @plsc.parallel_loop(0, L, unroll=2)
    def _pre(l):
        for g in range(NG):
            sl = pl.ds(g * LANES, LANES)
            idx_v[l, sl] = idx_v[l, sl] * tstride + lanes[g]

    def fire(c, s):
        pltpu.async_copy(
            pt_hbm.at[:, pl.ds(c * DC, DC), pl.ds(b0, NL)],
            slab_v.at[s], gsem.at[s])

    fire(0, 0)
    fire(1, 1)

    def chunk(c, s):
        pltpu.make_async_copy(
            pt_hbm.at[:, pl.ds(0, DC), pl.ds(b0, NL)],
            slab_v.at[s], gsem.at[s]).wait()

        @pl.when(c > 0)
        def _wait_out_free():
            pltpu.make_async_copy(
                out_v, ot_hbm.at[:, pl.ds(0, DC), pl.ds(b0, NL)],
                ssem).wait()

        @plsc.parallel_loop(0, L, unroll=4)
        def per_l(l):
            for g in range(NG):
                sl = pl.ds(g * LANES, LANES)
                sidx = idx_v[l, sl]
                for d in range(DC):
                    v = plsc.load_gather(
                        slab_v.at[s],
                        [zero, zero, sidx if d == 0 else sidx + d * dstride])
                    out_v[l, d, sl] = v
        pltpu.async_copy(out_v, ot_hbm.at[:, pl.ds(c * DC, DC), pl.ds(b0, NL)],
                         ssem)

        @pl.when(c + 2 < NDCH)
        def _refill():
            fire(c + 2, s)

    def group(g2, _):
        chunk(g2 * 2, 0)
        chunk(g2 * 2 + 1, 1)
        return _

    lax.fori_loop(0, NDCH // 2, group, None)
    pltpu.make_async_copy(
        out_v, ot_hbm.at[:, pl.ds(0, DC), pl.ds(b0, NL)], ssem).wait()


def kernel(params, indices):
    pt = params.transpose(1, 2, 0)
    it = indices.astype(jnp.int32).T

    mesh = plsc.VectorSubcoreMesh(core_axis_name="c", subcore_axis_name="s")
    k = pl.kernel(
        _body,
        mesh=mesh,
        out_type=jax.ShapeDtypeStruct((L, D, B), jnp.float32),
        scratch_types=[
            pltpu.VMEM((L, NL), jnp.int32),
            pltpu.VMEM((2, T, DC, NL), jnp.float32),
            pltpu.VMEM((L, DC, NL), jnp.float32),
            pltpu.SemaphoreType.DMA((2,)),
            pltpu.SemaphoreType.DMA,
        ],
        compiler_params=pltpu.CompilerParams(use_tc_tiling_on_sc=True,
                                             needs_layout_passes=False),
    )
    ot = k(pt, it)
    return ot.transpose(2, 0, 1)
